# Initial kernel scaffold; baseline (speedup 1.0000x reference)
import jax
import jax.numpy as jnp
from jax.experimental import pallas as pl


def kernel(x, W_embed):
    raise NotImplementedError("write your pallas kernel here")


# speedup vs baseline: 1.3069x; 1.3069x over previous
import functools

import jax
import jax.numpy as jnp
from jax import lax
from jax.experimental import pallas as pl
from jax.experimental.pallas import tpu as pltpu
from jax.experimental.pallas import tpu_sc as plsc

NC = 2
NS = 16
NW = NC * NS
CH = 128
D = 32


@functools.partial(jax.jit, static_argnums=(2,))
def _sc_gather(W_embed, idx, B):
    n_ch = B // (NW * CH)
    b_per_w = B // NW
    mesh = plsc.VectorSubcoreMesh(core_axis_name="c", subcore_axis_name="s")

    @functools.partial(
        pl.kernel,
        out_type=jax.ShapeDtypeStruct((B, D), jnp.float32),
        mesh=mesh,
        scratch_types=[
            pltpu.VMEM((n_ch, CH), jnp.int32),
            pltpu.VMEM((CH, D), jnp.float32),
            pltpu.SemaphoreType.DMA,
        ],
        compiler_params=pltpu.CompilerParams(use_tc_tiling_on_sc=False),
    )
    def gather_kernel(table_hbm, idx_hbm, out_hbm, idx_v, rows_v, gsem):
        wid = lax.axis_index("s") * NC + lax.axis_index("c")
        base = wid * b_per_w
        pltpu.sync_copy(idx_hbm.at[wid], idx_v)

        @pl.loop(0, n_ch)
        def _(j):
            pltpu.async_copy(table_hbm.at[idx_v.at[j]], rows_v, gsem).wait()
            pltpu.sync_copy(rows_v, out_hbm.at[pl.ds(base + j * CH, CH)])

    return gather_kernel(W_embed, idx.reshape(NW, n_ch, CH))


def kernel(x, W_embed):
    N, T = x.shape
    B = N * T
    idx = x.reshape(B).astype(jnp.int32)
    out = _sc_gather(W_embed, idx, B)
    return out.reshape(N, T, D)


# speedup vs baseline: 1.4973x; 1.1456x over previous
import functools

import jax
import jax.numpy as jnp
from jax import lax
from jax.experimental import pallas as pl
from jax.experimental.pallas import tpu as pltpu
from jax.experimental.pallas import tpu_sc as plsc

NC = 2
NS = 16
NW = NC * NS
CH = 128
D = 32
NBUF = 8


@functools.partial(jax.jit, static_argnums=(2,))
def _sc_gather(W_embed, idx, B):
    n_ch = B // (NW * CH)
    n_groups = n_ch // NBUF
    b_per_w = B // NW
    mesh = plsc.VectorSubcoreMesh(core_axis_name="c", subcore_axis_name="s")

    @functools.partial(
        pl.kernel,
        out_type=jax.ShapeDtypeStruct((B, D), jnp.float32),
        mesh=mesh,
        scratch_types=[
            pltpu.VMEM((n_ch, CH), jnp.int32),
            pltpu.VMEM((NBUF, CH, D), jnp.float32),
            pltpu.SemaphoreType.DMA((NBUF,)),
            pltpu.SemaphoreType.DMA((NBUF,)),
        ],
        compiler_params=pltpu.CompilerParams(use_tc_tiling_on_sc=False),
    )
    def gather_kernel(table_hbm, idx_hbm, out_hbm, idx_v, rows_v, gsem, ssem):
        wid = lax.axis_index("s") * NC + lax.axis_index("c")
        base = wid * b_per_w
        pltpu.sync_copy(idx_hbm.at[wid], idx_v)

        def g_copy(j, b):
            return pltpu.make_async_copy(
                table_hbm.at[idx_v.at[j]], rows_v.at[b], gsem.at[b])

        def s_copy(j, b):
            return pltpu.make_async_copy(
                rows_v.at[b], out_hbm.at[pl.ds(base + j * CH, CH)], ssem.at[b])

        for b in range(NBUF):
            g_copy(b, b).start()

        @pl.loop(0, n_groups)
        def _(g):
            j0 = g * NBUF
            for b in range(NBUF):
                g_copy(j0 + b, b).wait()
                s_copy(j0 + b, b).start()

            @pl.when(g + 1 < n_groups)
            def _():
                for b in range(NBUF):
                    s_copy(j0 + b, b).wait()
                    g_copy(j0 + NBUF + b, b).start()

        for b in range(NBUF):
            s_copy((n_groups - 1) * NBUF + b, b).wait()

    return gather_kernel(W_embed, idx.reshape(NW, n_ch, CH))


def kernel(x, W_embed):
    N, T = x.shape
    B = N * T
    idx = x.reshape(B).astype(jnp.int32)
    out = _sc_gather(W_embed, idx, B)
    return out.reshape(N, T, D)
